# baseline (speedup 1.0000x reference)
import functools

import jax
import jax.numpy as jnp
from jax import lax
from jax.experimental import pallas as pl
from jax.experimental.pallas import tpu as pltpu
from jax.experimental.pallas import tpu_sc as plsc

VOCAB = 1000000
EMB = 64
BATCH = 16384

NUM_CORES = 2
NUM_SUBCORES = 16
LANES = 16
NW = NUM_CORES * NUM_SUBCORES
B_PER_W = BATCH // NW
GCHUNK = 128
NCHUNK = B_PER_W // GCHUNK
GROUPS = B_PER_W // LANES


def _sc_body(u_hbm, v_hbm, table_hbm, out_hbm,
             idx_u, idx_v, u_rows, v_rows, out_v, sem):
    wid = lax.axis_index("s") * NUM_CORES + lax.axis_index("c")
    base = wid * B_PER_W

    pltpu.sync_copy(u_hbm.at[pl.ds(base, B_PER_W)], idx_u)
    pltpu.sync_copy(v_hbm.at[pl.ds(base, B_PER_W)], idx_v)

    copies = []
    for j in range(NCHUNK):
        sl = pl.ds(j * GCHUNK, GCHUNK)
        copies.append(pltpu.async_copy(
            table_hbm.at[idx_u.at[sl]], u_rows.at[sl], sem))
        copies.append(pltpu.async_copy(
            table_hbm.at[idx_v.at[sl]], v_rows.at[sl], sem))
    for c in copies:
        c.wait()

    lane = lax.iota(jnp.int32, LANES)
    perms = [lane ^ (1 << s) for s in range(4)]

    def group_body(g, _):
        acc = jnp.zeros((LANES,), jnp.float32)
        for l in range(LANES):
            r = g * LANES + l
            p = jnp.zeros((LANES,), jnp.float32)
            for c in range(EMB // LANES):
                cu = u_rows[r, pl.ds(c * LANES, LANES)]
                cv = v_rows[r, pl.ds(c * LANES, LANES)]
                p = p + cu * cv
            for s in range(4):
                p = p + p.at[perms[s]].get(mode="promise_in_bounds")
            acc = jnp.where(lane == l, p, acc)
        out_v[pl.ds(g * LANES, LANES)] = acc
        return 0

    lax.fori_loop(0, GROUPS, group_body, 0)

    pltpu.sync_copy(out_v, out_hbm.at[pl.ds(base, B_PER_W)])


@jax.jit
def _skipgram(u, v, emb_weight):
    mesh = plsc.VectorSubcoreMesh(core_axis_name="c", subcore_axis_name="s")
    return pl.kernel(
        _sc_body,
        out_type=jax.ShapeDtypeStruct((BATCH,), jnp.float32),
        mesh=mesh,
        compiler_params=pltpu.CompilerParams(use_tc_tiling_on_sc=False),
        scratch_types=[
            pltpu.VMEM((B_PER_W,), jnp.int32),
            pltpu.VMEM((B_PER_W,), jnp.int32),
            pltpu.VMEM((B_PER_W, EMB), jnp.float32),
            pltpu.VMEM((B_PER_W, EMB), jnp.float32),
            pltpu.VMEM((B_PER_W,), jnp.float32),
            pltpu.SemaphoreType.DMA,
        ],
    )(u, v, emb_weight)


def kernel(u, v, emb_weight):
    return _skipgram(u.astype(jnp.int32), v.astype(jnp.int32), emb_weight)
